# baseline (speedup 1.0000x reference)
import functools

import jax
import jax.numpy as jnp
from jax import lax
from jax.experimental import pallas as pl
from jax.experimental.pallas import tpu as pltpu
from jax.experimental.pallas import tpu_sc as plsc

_EMB = 512
_LANES = 16
_VPR = _EMB // _LANES
_GROUP = 32
_SUB = _GROUP // _LANES
_CAP = 16384


def _seg_max_body(n_valid, g_per_w, num_cores,
                  h_hbm, idx_hbm, batch_hbm, out_hbm,
                  batch_v, idx_v, rows0, rows1, stage_v, sem0, sem1):
  wid = lax.axis_index("s") * num_cores + lax.axis_index("c")
  gbase = (wid * g_per_w).astype(jnp.int32)

  pltpu.sync_copy(batch_hbm, batch_v.at[pl.ds(0, n_valid)])

  neg_inf = jnp.full((_LANES,), -jnp.inf, jnp.float32)

  def init_row(r, c):
    for j in range(_VPR):
      stage_v[r, pl.ds(j * _LANES, _LANES)] = neg_inf
    return c

  lax.fori_loop(0, g_per_w, init_row, 0)

  def bsearch(target):
    def step(_, lohi):
      lo, hi = lohi
      mid = (lo + hi) // 2
      v = batch_v[pl.ds(mid, _LANES)][0]
      return (jnp.where(v < target, mid + 1, lo),
              jnp.where(v < target, hi, mid))

    lo, _ = lax.fori_loop(0, 17, step, (jnp.int32(0), jnp.int32(n_valid)))
    return lo

  s_w = bsearch(gbase)
  e_w = bsearch(gbase + g_per_w)

  base = (s_w // 8) * 8
  span = e_w - base
  n_super = (span + _CAP - 1) // _CAP

  bufs = ((rows0, sem0), (rows1, sem1))

  def flush(cur_g, acc):
    grel = cur_g - gbase

    @pl.when((grel >= 0) & (grel < g_per_w))
    def _():
      for j in range(_VPR):
        sl = pl.ds(j * _LANES, _LANES)
        stage_v[grel, sl] = jnp.maximum(stage_v[grel, sl], acc[j])

  def do_super(k, carry):
    sstart = base + k * _CAP
    ck = jnp.minimum(sstart, n_valid - (_CAP + _GROUP))
    pltpu.sync_copy(idx_hbm.at[pl.ds(ck, _CAP + _GROUP)], idx_v)
    rem = span - k * _CAP
    tk = jnp.clip((rem + _GROUP - 1) // _GROUP, 1, _CAP // _GROUP)
    npairs = (tk + 1) // 2
    n_proc = npairs * 2

    def gstart(g):
      return jnp.minimum(sstart + g * _GROUP, n_valid - _GROUP)

    def issue(g, rows, sem):
      idx_ref = idx_v.at[pl.ds(gstart(g) - ck, _GROUP)]
      pltpu.make_async_copy(h_hbm.at[idx_ref], rows, sem).start()

    issue(0, rows0, sem0)

    def do_pair(p, carry2):
      for b in range(2):
        rows, sem = bufs[b]
        nrows, nsem = bufs[1 - b]
        g = 2 * p + b
        pltpu.make_async_copy(
            h_hbm.at[idx_v.at[pl.ds(0, _GROUP)]], rows, sem).wait()

        @pl.when(g + 1 < n_proc)
        def _():
          issue(g + 1, nrows, nsem)

        p0 = gstart(g)

        def do_sub(s, carry3):
          cur_g, acc = carry3
          gvec = batch_v[pl.ds(p0 + s * _LANES, _LANES)]
          rbase = s * _LANES
          for r in range(_LANES):
            gr = gvec[r]
            changed = gr != cur_g

            @pl.when(changed)
            def _():
              flush(cur_g, acc)

            row = [rows[rbase + r, pl.ds(j * _LANES, _LANES)]
                   for j in range(_VPR)]
            acc = [jnp.where(changed, row[j], jnp.maximum(acc[j], row[j]))
                   for j in range(_VPR)]
            cur_g = gr
          return cur_g, acc

        carry2 = lax.fori_loop(0, _SUB, do_sub, carry2)
      return carry2

    carry = lax.fori_loop(0, npairs, do_pair, carry)
    return carry

  carry0 = (jnp.int32(-1), [neg_inf] * _VPR)
  cur_g, acc = lax.fori_loop(0, n_super, do_super, carry0)
  flush(cur_g, acc)

  pltpu.sync_copy(stage_v, out_hbm.at[pl.ds(gbase, g_per_w)])


@jax.jit
def kernel(h, indices, batch):
  n_nodes, emb = h.shape
  n_valid = indices.shape[0]
  n_graphs = 1024
  info = plsc.get_sparse_core_info()
  nc, ns = info.num_cores, info.num_subcores
  g_per_w = n_graphs // (nc * ns)
  mesh = plsc.VectorSubcoreMesh(core_axis_name="c", subcore_axis_name="s",
                                num_cores=nc, num_subcores=ns)
  body = functools.partial(_seg_max_body, n_valid, g_per_w, nc)
  run = pl.kernel(
      body,
      out_type=jax.ShapeDtypeStruct((n_graphs, emb), jnp.float32),
      mesh=mesh,
      compiler_params=pltpu.CompilerParams(use_tc_tiling_on_sc=False),
      scratch_types=[
          pltpu.VMEM((n_valid + _LANES,), jnp.int32),
          pltpu.VMEM((_CAP + _GROUP,), jnp.int32),
          pltpu.VMEM((_GROUP, emb), jnp.float32),
          pltpu.VMEM((_GROUP, emb), jnp.float32),
          pltpu.VMEM((g_per_w, emb), jnp.float32),
          pltpu.SemaphoreType.DMA,
          pltpu.SemaphoreType.DMA,
      ],
  )
  return run(h.reshape(-1, emb), indices, batch)


# speedup vs baseline: 3.4011x; 3.4011x over previous
import functools

import jax
import jax.numpy as jnp
from jax import lax
from jax.experimental import pallas as pl
from jax.experimental.pallas import tpu as pltpu
from jax.experimental.pallas import tpu_sc as plsc

_EMB = 512
_LANES = 16
_VPR = _EMB // _LANES
_GROUP = 32
_SUB = _GROUP // _LANES
_CAP = 16384


def _seg_max_body(n_valid, g_per_w, num_cores,
                  h_hbm, idx_hbm, batch_hbm, out_hbm,
                  batch_v, idx_v, rows0, rows1, stage_v, gid_s, sem0, sem1):
  wid = lax.axis_index("s") * num_cores + lax.axis_index("c")
  gbase = (wid * g_per_w).astype(jnp.int32)

  pltpu.sync_copy(batch_hbm, batch_v.at[pl.ds(0, n_valid)])

  neg_inf = jnp.full((_LANES,), -jnp.inf, jnp.float32)

  def init_row(r, c):
    for j in range(_VPR):
      stage_v[r, pl.ds(j * _LANES, _LANES)] = neg_inf
    return c

  lax.fori_loop(0, g_per_w, init_row, 0)

  def bsearch(target):
    def step(_, lohi):
      lo, hi = lohi
      mid = (lo + hi) // 2
      v = batch_v[pl.ds(mid, _LANES)][0]
      return (jnp.where(v < target, mid + 1, lo),
              jnp.where(v < target, hi, mid))

    lo, _ = lax.fori_loop(0, 17, step, (jnp.int32(0), jnp.int32(n_valid)))
    return lo

  s_w = bsearch(gbase)
  e_w = bsearch(gbase + g_per_w)

  base = (s_w // 8) * 8
  span = e_w - base
  n_super = (span + _CAP - 1) // _CAP

  bufs = ((rows0, sem0), (rows1, sem1))

  def flush(cur_g, acc):
    grel = cur_g - gbase

    @pl.when((grel >= 0) & (grel < g_per_w))
    def _():
      for j in range(_VPR):
        sl = pl.ds(j * _LANES, _LANES)
        stage_v[grel, sl] = jnp.maximum(stage_v[grel, sl], acc[j])

  def do_super(k, carry):
    sstart = base + k * _CAP
    ck = jnp.minimum(sstart, n_valid - (_CAP + _GROUP))
    pltpu.sync_copy(idx_hbm.at[pl.ds(ck, _CAP + _GROUP)], idx_v)
    rem = span - k * _CAP
    tk = jnp.clip((rem + _GROUP - 1) // _GROUP, 1, _CAP // _GROUP)
    npairs = (tk + 1) // 2
    n_proc = npairs * 2

    def gstart(g):
      return jnp.minimum(sstart + g * _GROUP, n_valid - _GROUP)

    def issue(g, rows, sem):
      idx_ref = idx_v.at[pl.ds(gstart(g) - ck, _GROUP)]
      pltpu.make_async_copy(h_hbm.at[idx_ref], rows, sem).start()

    issue(0, rows0, sem0)

    def do_pair(p, carry2):
      for b in range(2):
        rows, sem = bufs[b]
        nrows, nsem = bufs[1 - b]
        g = 2 * p + b
        pltpu.make_async_copy(
            h_hbm.at[idx_v.at[pl.ds(0, _GROUP)]], rows, sem).wait()

        @pl.when(g + 1 < n_proc)
        def _():
          issue(g + 1, nrows, nsem)

        p0 = gstart(g)

        for s in range(_SUB):
          gvec = batch_v[pl.ds(p0 + s * _LANES, _LANES)]
          for r in range(_LANES):
            gid_s[s * _LANES + r] = gvec[r]

        def do_row(r, carry3):
          cur_g, acc = carry3
          gr = gid_s[r]
          changed = gr != cur_g

          @pl.when(changed)
          def _():
            flush(cur_g, acc)

          row = [rows[r, pl.ds(j * _LANES, _LANES)] for j in range(_VPR)]
          acc = [jnp.where(changed, row[j], jnp.maximum(acc[j], row[j]))
                 for j in range(_VPR)]
          return gr, acc

        carry2 = lax.fori_loop(0, _GROUP, do_row, carry2)
      return carry2

    carry = lax.fori_loop(0, npairs, do_pair, carry)
    return carry

  carry0 = (jnp.int32(-1), [neg_inf] * _VPR)
  cur_g, acc = lax.fori_loop(0, n_super, do_super, carry0)
  flush(cur_g, acc)

  pltpu.sync_copy(stage_v, out_hbm.at[pl.ds(gbase, g_per_w)])


@jax.jit
def kernel(h, indices, batch):
  n_nodes, emb = h.shape
  n_valid = indices.shape[0]
  n_graphs = 1024
  info = plsc.get_sparse_core_info()
  nc, ns = info.num_cores, info.num_subcores
  g_per_w = n_graphs // (nc * ns)
  mesh = plsc.VectorSubcoreMesh(core_axis_name="c", subcore_axis_name="s",
                                num_cores=nc, num_subcores=ns)
  body = functools.partial(_seg_max_body, n_valid, g_per_w, nc)
  run = pl.kernel(
      body,
      out_type=jax.ShapeDtypeStruct((n_graphs, emb), jnp.float32),
      mesh=mesh,
      scratch_types=[
          pltpu.VMEM((n_valid + _LANES,), jnp.int32),
          pltpu.VMEM((_CAP + _GROUP,), jnp.int32),
          pltpu.VMEM((_GROUP, emb), jnp.float32),
          pltpu.VMEM((_GROUP, emb), jnp.float32),
          pltpu.VMEM((g_per_w, emb), jnp.float32),
          pltpu.SMEM((_GROUP,), jnp.int32),
          pltpu.SemaphoreType.DMA,
          pltpu.SemaphoreType.DMA,
      ],
  )
  return run(h.reshape(-1, emb), indices, batch)


# speedup vs baseline: 3.7352x; 1.0982x over previous
import functools

import jax
import jax.numpy as jnp
from jax import lax
from jax.experimental import pallas as pl
from jax.experimental.pallas import tpu as pltpu
from jax.experimental.pallas import tpu_sc as plsc

_EMB = 512
_LANES = 16
_VPR = _EMB // _LANES
_GROUP = 32
_SUB = _GROUP // _LANES
_CAP = 8192


def _seg_max_body(n_valid, g_per_w, num_cores,
                  h_hbm, idx_hbm, batch_hbm, out_hbm,
                  batch_v, idx_v, rows0, rows1, rows2, stage_v, gid_s,
                  sem0, sem1, sem2):
  wid = lax.axis_index("s") * num_cores + lax.axis_index("c")
  gbase = (wid * g_per_w).astype(jnp.int32)

  pltpu.sync_copy(batch_hbm, batch_v.at[pl.ds(0, n_valid)])

  neg_inf = jnp.full((_LANES,), -jnp.inf, jnp.float32)

  def init_row(r, c):
    for j in range(_VPR):
      stage_v[r, pl.ds(j * _LANES, _LANES)] = neg_inf
    return c

  lax.fori_loop(0, g_per_w, init_row, 0)

  def bsearch(target):
    def step(_, lohi):
      lo, hi = lohi
      mid = (lo + hi) // 2
      v = batch_v[pl.ds(mid, _LANES)][0]
      return (jnp.where(v < target, mid + 1, lo),
              jnp.where(v < target, hi, mid))

    lo, _ = lax.fori_loop(0, 17, step, (jnp.int32(0), jnp.int32(n_valid)))
    return lo

  s_w = bsearch(gbase)
  e_w = bsearch(gbase + g_per_w)

  base = (s_w // 8) * 8
  span = e_w - base
  n_super = (span + _CAP - 1) // _CAP

  bufs = ((rows0, sem0), (rows1, sem1), (rows2, sem2))

  def flush(cur_g, acc):
    grel = cur_g - gbase

    @pl.when((grel >= 0) & (grel < g_per_w))
    def _():
      for j in range(_VPR):
        sl = pl.ds(j * _LANES, _LANES)
        stage_v[grel, sl] = jnp.maximum(stage_v[grel, sl], acc[j])

  def do_super(k, carry):
    sstart = base + k * _CAP
    ck = jnp.minimum(sstart, n_valid - (_CAP + _GROUP))
    pltpu.sync_copy(idx_hbm.at[pl.ds(ck, _CAP + _GROUP)], idx_v)
    rem = span - k * _CAP
    tk = jnp.clip((rem + _GROUP - 1) // _GROUP, 1, _CAP // _GROUP)
    ntrip = (tk + 2) // 3
    n_proc = ntrip * 3

    def gstart(g):
      return jnp.minimum(sstart + g * _GROUP, n_valid - _GROUP)

    def issue(g, rows, sem):
      idx_ref = idx_v.at[pl.ds(gstart(g) - ck, _GROUP)]
      pltpu.make_async_copy(h_hbm.at[idx_ref], rows, sem).start()

    issue(0, rows0, sem0)
    issue(1, rows1, sem1)

    def do_pair(p, carry2):
      for b in range(3):
        rows, sem = bufs[b]
        nrows, nsem = bufs[(b + 2) % 3]
        g = 3 * p + b
        pltpu.make_async_copy(
            h_hbm.at[idx_v.at[pl.ds(0, _GROUP)]], rows, sem).wait()

        @pl.when(g + 2 < n_proc)
        def _():
          issue(g + 2, nrows, nsem)

        p0 = gstart(g)

        for s in range(_SUB):
          gvec = batch_v[pl.ds(p0 + s * _LANES, _LANES)]
          for r in range(_LANES):
            gid_s[s * _LANES + r] = gvec[r]

        def do_row(r, carry3):
          cur_g, acc = carry3
          gr = gid_s[r]
          changed = gr != cur_g

          @pl.when(changed)
          def _():
            flush(cur_g, acc)

          row = [rows[r, pl.ds(j * _LANES, _LANES)] for j in range(_VPR)]
          acc = [jnp.where(changed, row[j], jnp.maximum(acc[j], row[j]))
                 for j in range(_VPR)]
          return gr, acc

        carry2 = lax.fori_loop(0, _GROUP, do_row, carry2)
      return carry2

    carry = lax.fori_loop(0, ntrip, do_pair, carry)
    return carry

  carry0 = (jnp.int32(-1), [neg_inf] * _VPR)
  cur_g, acc = lax.fori_loop(0, n_super, do_super, carry0)
  flush(cur_g, acc)

  pltpu.sync_copy(stage_v, out_hbm.at[pl.ds(gbase, g_per_w)])


@jax.jit
def kernel(h, indices, batch):
  n_nodes, emb = h.shape
  n_valid = indices.shape[0]
  n_graphs = 1024
  info = plsc.get_sparse_core_info()
  nc, ns = info.num_cores, info.num_subcores
  g_per_w = n_graphs // (nc * ns)
  mesh = plsc.VectorSubcoreMesh(core_axis_name="c", subcore_axis_name="s",
                                num_cores=nc, num_subcores=ns)
  body = functools.partial(_seg_max_body, n_valid, g_per_w, nc)
  run = pl.kernel(
      body,
      out_type=jax.ShapeDtypeStruct((n_graphs, emb), jnp.float32),
      mesh=mesh,
      scratch_types=[
          pltpu.VMEM((n_valid + _LANES,), jnp.int32),
          pltpu.VMEM((_CAP + _GROUP,), jnp.int32),
          pltpu.VMEM((_GROUP, emb), jnp.float32),
          pltpu.VMEM((_GROUP, emb), jnp.float32),
          pltpu.VMEM((_GROUP, emb), jnp.float32),
          pltpu.VMEM((g_per_w, emb), jnp.float32),
          pltpu.SMEM((_GROUP,), jnp.int32),
          pltpu.SemaphoreType.DMA,
          pltpu.SemaphoreType.DMA,
          pltpu.SemaphoreType.DMA,
      ],
  )
  return run(h.reshape(-1, emb), indices, batch)
